# baseline (speedup 1.0000x reference)
import functools
import math

import jax
import jax.numpy as jnp
from jax import lax
from jax.experimental import pallas as pl
from jax.experimental.pallas import tpu as pltpu
from jax.experimental.pallas import tpu_sc as plsc

D_MODEL = 64
LANES = 16
NUM_CORES = 2
NUM_SUBCORES = 16
NUM_WORKERS = NUM_CORES * NUM_SUBCORES
CHUNK = 128
NSLOTS = 4
ROWS_PER_ITER = 8
SCALE = math.sqrt(D_MODEL)


def _sc_embed(idx, table, n_chunks):
    mesh = plsc.VectorSubcoreMesh(core_axis_name="c", subcore_axis_name="s")

    @functools.partial(
        pl.kernel,
        mesh=mesh,
        out_type=jax.ShapeDtypeStruct(
            (NUM_WORKERS, n_chunks, CHUNK, D_MODEL), jnp.float32),
        scratch_types=[
            pltpu.VMEM((n_chunks, CHUNK), jnp.int32),
            pltpu.VMEM((NSLOTS, CHUNK, D_MODEL), jnp.float32),
            pltpu.SemaphoreType.DMA,
            pltpu.SemaphoreType.DMA,
        ],
        compiler_params=pltpu.CompilerParams(use_tc_tiling_on_sc=False),
    )
    def k(idx_hbm, table_hbm, out_hbm, idx_v, rows_v, gsem, osem):
        wid = lax.axis_index("s") * NUM_CORES + lax.axis_index("c")
        pltpu.sync_copy(idx_hbm.at[wid], idx_v)

        def gather(j, slot):
            return pltpu.async_copy(
                table_hbm.at[idx_v.at[j]], rows_v.at[slot], gsem)

        gather(0, 0)
        gather(1, 1)

        def scale_slot(slot):
            def row_body(r, _):
                for rr in range(ROWS_PER_ITER):
                    for c in range(D_MODEL // LANES):
                        sl = pl.ds(c * LANES, LANES)
                        v = rows_v[slot, r + rr, sl]
                        rows_v[slot, r + rr, sl] = v * SCALE
                return _
            lax.fori_loop(0, CHUNK // ROWS_PER_ITER,
                          lambda i, _: row_body(i * ROWS_PER_ITER, _), 0)

        def outer(i, _):
            for bb in range(NSLOTS):
                j = i * NSLOTS + bb
                pltpu.make_async_copy(
                    table_hbm.at[idx_v.at[j]], rows_v.at[bb], gsem).wait()
                scale_slot(bb)
                pltpu.async_copy(rows_v.at[bb], out_hbm.at[wid, j], osem)
                @pl.when(j >= 2)
                def _wait_out():
                    pltpu.make_async_copy(
                        rows_v.at[bb], out_hbm.at[wid, j], osem).wait()

                @pl.when(j + 2 < n_chunks)
                def _next_gather():
                    gather(j + 2, (bb + 2) % NSLOTS)
            return _

        lax.fori_loop(0, n_chunks // NSLOTS, outer, 0)

        for _ in range(2):
            pltpu.make_async_copy(
                rows_v.at[0], out_hbm.at[wid, 0], osem).wait()

    return k(idx, table)


def kernel(x, emb_weight):
    batch, hist = x.shape
    rows = batch * hist
    per_w = rows // NUM_WORKERS
    n_chunks = per_w // CHUNK
    idx = x.reshape(NUM_WORKERS, n_chunks, CHUNK).astype(jnp.int32)
    out = _sc_embed(idx, emb_weight, n_chunks)
    return out.reshape(batch, hist, D_MODEL)
